# Initial kernel scaffold; baseline (speedup 1.0000x reference)
import jax
import jax.numpy as jnp
from jax.experimental import pallas as pl


def kernel(x_user, x_movie, edge_index_watched, edge_index_rev, edge_label_index, Wl_um, Wr_um, b_um, Wl_mu, Wr_mu, b_mu, Wl2, Wr2, b2, Wl3, Wr3, b3):
    raise NotImplementedError("write your pallas kernel here")


# speedup vs baseline: 1.9551x; 1.9551x over previous
import functools

import jax
import jax.numpy as jnp
from jax import lax
from jax.experimental import pallas as pl
from jax.experimental.pallas import tpu as pltpu
from jax.experimental.pallas import tpu_sc as plsc

N = 10000
NP = 10240
NS = 16
NC = 2
KE = 80
RPT = NP // NS


def _sc_mesh():
    return plsc.VectorSubcoreMesh(core_axis_name="c", subcore_axis_name="s")


@functools.cache
def _make_segsum_edge_split(E):
    nch = E // (NC * NS * KE)

    @functools.partial(
        pl.kernel,
        out_type=(jax.ShapeDtypeStruct((NP, 128), jnp.float32),
                  jax.ShapeDtypeStruct((NP, 128), jnp.float32)),
        mesh=_sc_mesh(),
        scratch_types=[
            pltpu.VMEM((KE,), jnp.int32),
            pltpu.VMEM((KE,), jnp.int32),
            pltpu.VMEM((KE, 128), jnp.float32),
            pltpu.VMEM_SHARED((NP, 128), jnp.float32),
            pltpu.SemaphoreType.DMA,
        ],
    )
    def seg(x, src3, dst3, zrows, out0, out1, idx_s, idx_d, rows, acc, sem):
        c = lax.axis_index("c")
        s = lax.axis_index("s")
        w = c * NS + s
        pltpu.sync_copy(zrows, acc.at[pl.ds(s * RPT, RPT)])
        plsc.subcore_barrier()

        def body(ch, carry):
            pltpu.sync_copy(src3.at[w, ch], idx_s)
            pltpu.sync_copy(dst3.at[w, ch], idx_d)
            pltpu.async_copy(x.at[idx_s], rows, sem).wait()
            pltpu.sync_copy(rows, acc.at[idx_d], add=True)
            return carry

        lax.fori_loop(0, nch, body, 0)
        plsc.subcore_barrier()

        @pl.when(c == 0)
        def _():
            pltpu.sync_copy(acc.at[pl.ds(s * RPT, RPT)], out0.at[pl.ds(s * RPT, RPT)])

        @pl.when(c == 1)
        def _():
            pltpu.sync_copy(acc.at[pl.ds(s * RPT, RPT)], out1.at[pl.ds(s * RPT, RPT)])

    return seg


@functools.cache
def _make_segsum_feat_split(E):
    nch = E // (NS * KE)

    @functools.partial(
        pl.kernel,
        out_type=(jax.ShapeDtypeStruct((NP, 128), jnp.float32),
                  jax.ShapeDtypeStruct((NP, 128), jnp.float32)),
        mesh=_sc_mesh(),
        scratch_types=[
            pltpu.VMEM((KE,), jnp.int32),
            pltpu.VMEM((KE,), jnp.int32),
            pltpu.VMEM((KE, 128), jnp.float32),
            pltpu.VMEM_SHARED((NP, 128), jnp.float32),
            pltpu.SemaphoreType.DMA,
        ],
    )
    def seg(xa, xb, src3, dst3, zrows, outa, outb, idx_s, idx_d, rows, acc, sem):
        c = lax.axis_index("c")
        s = lax.axis_index("s")
        pltpu.sync_copy(zrows, acc.at[pl.ds(s * RPT, RPT)])
        plsc.subcore_barrier()

        def body(ch, carry):
            pltpu.sync_copy(src3.at[s, ch], idx_s)
            pltpu.sync_copy(dst3.at[s, ch], idx_d)

            @pl.when(c == 0)
            def _():
                pltpu.async_copy(xa.at[idx_s], rows, sem).wait()

            @pl.when(c == 1)
            def _():
                pltpu.async_copy(xb.at[idx_s], rows, sem).wait()

            pltpu.sync_copy(rows, acc.at[idx_d], add=True)
            return carry

        lax.fori_loop(0, nch, body, 0)
        plsc.subcore_barrier()

        @pl.when(c == 0)
        def _():
            pltpu.sync_copy(acc.at[pl.ds(s * RPT, RPT)], outa.at[pl.ds(s * RPT, RPT)])

        @pl.when(c == 1)
        def _():
            pltpu.sync_copy(acc.at[pl.ds(s * RPT, RPT)], outb.at[pl.ds(s * RPT, RPT)])

    return seg


@functools.cache
def _make_counts(E):
    nch = E // (NS * KE)

    @functools.partial(
        pl.kernel,
        out_type=(jax.ShapeDtypeStruct((NP, 128), jnp.float32),
                  jax.ShapeDtypeStruct((NP, 128), jnp.float32)),
        mesh=_sc_mesh(),
        scratch_types=[
            pltpu.VMEM((KE,), jnp.int32),
            pltpu.VMEM((KE, 128), jnp.float32),
            pltpu.VMEM_SHARED((NP, 128), jnp.float32),
        ],
    )
    def cnt(dw3, dr3, ones_hbm, zrows, outw, outr, idx_d, ones_v, acc):
        c = lax.axis_index("c")
        s = lax.axis_index("s")
        pltpu.sync_copy(ones_hbm, ones_v)
        pltpu.sync_copy(zrows, acc.at[pl.ds(s * RPT, RPT)])
        plsc.subcore_barrier()

        def body(ch, carry):
            @pl.when(c == 0)
            def _():
                pltpu.sync_copy(dw3.at[s, ch], idx_d)

            @pl.when(c == 1)
            def _():
                pltpu.sync_copy(dr3.at[s, ch], idx_d)

            pltpu.sync_copy(ones_v, acc.at[idx_d], add=True)
            return carry

        lax.fori_loop(0, nch, body, 0)
        plsc.subcore_barrier()

        @pl.when(c == 0)
        def _():
            pltpu.sync_copy(acc.at[pl.ds(s * RPT, RPT)], outw.at[pl.ds(s * RPT, RPT)])

        @pl.when(c == 1)
        def _():
            pltpu.sync_copy(acc.at[pl.ds(s * RPT, RPT)], outr.at[pl.ds(s * RPT, RPT)])

    return cnt


@functools.cache
def _make_decoder_gather(BP, KD):
    per_tile = BP // (NC * NS)
    nch = per_tile // KD

    @functools.partial(
        pl.kernel,
        out_type=tuple(jax.ShapeDtypeStruct((BP, 128), jnp.float32) for _ in range(4)),
        mesh=_sc_mesh(),
        scratch_types=[
            pltpu.VMEM((KD,), jnp.int32),
            pltpu.VMEM((KD, 128), jnp.float32),
            pltpu.SemaphoreType.DMA,
        ],
    )
    def dec(hua, hub, hma, hmb, u3, m3, gua, gub, gma, gmb, idx, rows, sem):
        c = lax.axis_index("c")
        s = lax.axis_index("s")
        w = c * NS + s

        def body(ch, carry):
            base = w * per_tile + ch * KD
            pltpu.sync_copy(u3.at[w, ch], idx)
            pltpu.async_copy(hua.at[idx], rows, sem).wait()
            pltpu.sync_copy(rows, gua.at[pl.ds(base, KD)])
            pltpu.async_copy(hub.at[idx], rows, sem).wait()
            pltpu.sync_copy(rows, gub.at[pl.ds(base, KD)])
            pltpu.sync_copy(m3.at[w, ch], idx)
            pltpu.async_copy(hma.at[idx], rows, sem).wait()
            pltpu.sync_copy(rows, gma.at[pl.ds(base, KD)])
            pltpu.async_copy(hmb.at[idx], rows, sem).wait()
            pltpu.sync_copy(rows, gmb.at[pl.ds(base, KD)])
            return carry

        lax.fori_loop(0, nch, body, 0)

    return dec


def _dot():
    return functools.partial(
        lax.dot_general,
        dimension_numbers=(((1,), (0,)), ((), ())),
        preferred_element_type=jnp.float32,
        precision=lax.Precision.HIGHEST,
    )


def _finish(res, bias, oa, ob):
    res += bias[...]
    res = jnp.where(res >= 0.0, res, 0.01 * res)
    oa[...] = res[:, :128]
    ob[...] = res[:, 128:]


def _sage1_body(s0, s1, cnt, xd, wl, wr, bias, oa, ob):
    dot = _dot()
    inv = 1.0 / jnp.maximum(cnt[:, 0:1], 1.0)
    mean = (s0[...] + s1[...]) * inv
    res = dot(mean, wl[...]) + dot(xd[...], wr[...])
    _finish(res, bias, oa, ob)


def _sage2_body(sa, sb, cnt, xda, xdb, wla, wlb, wra, wrb, bias, oa, ob):
    dot = _dot()
    inv = 1.0 / jnp.maximum(cnt[:, 0:1], 1.0)
    res = dot(sa[...] * inv, wla[...])
    res += dot(sb[...] * inv, wlb[...])
    res += dot(xda[...], wra[...])
    res += dot(xdb[...], wrb[...])
    _finish(res, bias, oa, ob)


RB = 1024
_row = lambda i: (i, 0)
_full = lambda i: (0, 0)


@functools.cache
def _make_sage1():
    return pl.pallas_call(
        _sage1_body,
        grid=(NP // RB,),
        in_specs=[
            pl.BlockSpec((RB, 128), _row),
            pl.BlockSpec((RB, 128), _row),
            pl.BlockSpec((RB, 128), _row),
            pl.BlockSpec((RB, 128), _row),
            pl.BlockSpec((128, 256), _full),
            pl.BlockSpec((128, 256), _full),
            pl.BlockSpec((1, 256), _full),
        ],
        out_specs=[pl.BlockSpec((RB, 128), _row)] * 2,
        out_shape=[jax.ShapeDtypeStruct((NP, 128), jnp.float32)] * 2,
    )


@functools.cache
def _make_sage2():
    return pl.pallas_call(
        _sage2_body,
        grid=(NP // RB,),
        in_specs=[
            pl.BlockSpec((RB, 128), _row),
            pl.BlockSpec((RB, 128), _row),
            pl.BlockSpec((RB, 128), _row),
            pl.BlockSpec((RB, 128), _row),
            pl.BlockSpec((RB, 128), _row),
            pl.BlockSpec((128, 256), _full),
            pl.BlockSpec((128, 256), _full),
            pl.BlockSpec((128, 256), _full),
            pl.BlockSpec((128, 256), _full),
            pl.BlockSpec((1, 256), _full),
        ],
        out_specs=[pl.BlockSpec((RB, 128), _row)] * 2,
        out_shape=[jax.ShapeDtypeStruct((NP, 128), jnp.float32)] * 2,
    )


def _dot_body(gua, gub, gma, gmb, o):
    o[...] = jnp.sum(gua[...] * gma[...] + gub[...] * gmb[...], axis=1)


@functools.cache
def _make_pair_dot(BP):
    return pl.pallas_call(
        _dot_body,
        grid=(BP // RB,),
        in_specs=[pl.BlockSpec((RB, 128), _row)] * 4,
        out_specs=pl.BlockSpec((RB,), lambda i: (i,)),
        out_shape=jax.ShapeDtypeStruct((BP,), jnp.float32),
    )


def kernel(x_user, x_movie, edge_index_watched, edge_index_rev, edge_label_index,
           Wl_um, Wr_um, b_um, Wl_mu, Wr_mu, b_mu,
           Wl2, Wr2, b2, Wl3, Wr3, b3):
    f32 = jnp.float32
    E = edge_index_watched.shape[1]
    nch1 = E // (NC * NS * KE)
    nch2 = E // (NS * KE)
    assert E == NC * NS * nch1 * KE and E == NS * nch2 * KE

    ew = edge_index_watched.astype(jnp.int32)
    er = edge_index_rev.astype(jnp.int32)
    srcw1 = ew[0].reshape(NC * NS, nch1, KE)
    dstw1 = ew[1].reshape(NC * NS, nch1, KE)
    srcr1 = er[0].reshape(NC * NS, nch1, KE)
    dstr1 = er[1].reshape(NC * NS, nch1, KE)
    srcw2 = ew[0].reshape(NS, nch2, KE)
    dstw2 = ew[1].reshape(NS, nch2, KE)
    srcr2 = er[0].reshape(NS, nch2, KE)
    dstr2 = er[1].reshape(NS, nch2, KE)

    zr128 = jnp.zeros((RPT, 128), f32)
    ones128 = jnp.ones((KE, 128), f32)

    counts_w, counts_r = _make_counts(E)(dstw2, dstr2, ones128, zr128)

    seg1 = _make_segsum_edge_split(E)
    seg2 = _make_segsum_feat_split(E)
    sage1 = _make_sage1()
    sage2 = _make_sage2()

    sm0, sm1 = seg1(x_user, srcw1, dstw1, zr128)
    su0, su1 = seg1(x_movie, srcr1, dstr1, zr128)
    hm_a, hm_b = sage1(sm0, sm1, counts_w, x_movie, Wl_um, Wr_um, b_um.reshape(1, 256))
    hu_a, hu_b = sage1(su0, su1, counts_r, x_user, Wl_mu, Wr_mu, b_mu.reshape(1, 256))

    sm_a, sm_b = seg2(hu_a, hu_b, srcw2, dstw2, zr128)
    su_a, su_b = seg2(hm_a, hm_b, srcr2, dstr2, zr128)
    hm2_a, hm2_b = sage2(sm_a, sm_b, counts_w, hm_a, hm_b,
                         Wl2[:128], Wl2[128:], Wr2[:128], Wr2[128:],
                         b2.reshape(1, 256))
    hu2_a, hu2_b = sage2(su_a, su_b, counts_r, hu_a, hu_b,
                         Wl2[:128], Wl2[128:], Wr2[:128], Wr2[128:],
                         b2.reshape(1, 256))

    sm_a, sm_b = seg2(hu2_a, hu2_b, srcw2, dstw2, zr128)
    su_a, su_b = seg2(hm2_a, hm2_b, srcr2, dstr2, zr128)
    hm3_a, hm3_b = sage2(sm_a, sm_b, counts_w, hm2_a, hm2_b,
                         Wl3[:128], Wl3[128:], Wr3[:128], Wr3[128:],
                         b3.reshape(1, 256))
    hu3_a, hu3_b = sage2(su_a, su_b, counts_r, hu2_a, hu2_b,
                         Wl3[:128], Wl3[128:], Wr3[:128], Wr3[128:],
                         b3.reshape(1, 256))

    B = edge_label_index.shape[1]
    KD = 80
    BP = -(-B // (NC * NS * KD)) * (NC * NS * KD)
    eli = edge_label_index.astype(jnp.int32)
    u3 = jnp.pad(eli[0], (0, BP - B)).reshape(NC * NS, -1, KD)
    m3 = jnp.pad(eli[1], (0, BP - B)).reshape(NC * NS, -1, KD)
    gua, gub, gma, gmb = _make_decoder_gather(BP, KD)(hu3_a, hu3_b, hm3_a, hm3_b, u3, m3)
    dots = _make_pair_dot(BP)(gua, gub, gma, gmb)
    return dots[:B]


# speedup vs baseline: 3.6700x; 1.8772x over previous
import functools

import jax
import jax.numpy as jnp
from jax import lax
from jax.experimental import pallas as pl
from jax.experimental.pallas import tpu as pltpu
from jax.experimental.pallas import tpu_sc as plsc

N = 10000
NP = 10240
NS = 16
NC = 2
KE = 80
RPT = NP // NS
ZROW = N


def _sc_mesh():
    return plsc.VectorSubcoreMesh(core_axis_name="c", subcore_axis_name="s")


def _seg_pipeline(xs, e3_tile, acc, ibufs, rows, isems, gsems, ssems, nchp):
    def idx_load(ch, sl):
        pltpu.async_copy(e3_tile.at[ch], ibufs[sl], isems[sl])

    def idx_wait(ch, sl):
        pltpu.make_async_copy(e3_tile.at[ch], ibufs[sl], isems[sl]).wait()

    def gissue(rb, isl):
        for pred, x in xs:
            @pl.when(pred)
            def _():
                pltpu.async_copy(x.at[ibufs[isl].at[0]], rows[rb], gsems[rb])

    def gwait(rb, isl):
        for pred, x in xs:
            @pl.when(pred)
            def _():
                pltpu.make_async_copy(x.at[ibufs[isl].at[0]], rows[rb],
                                      gsems[rb]).wait()

    def sissue(rb, isl):
        pltpu.async_copy(rows[rb], acc.at[ibufs[isl].at[1]], ssems[rb],
                         add=True)

    def swait(rb):
        pltpu.make_async_copy(rows[rb], acc.at[ibufs[0].at[1]],
                              ssems[rb]).wait()

    def chunk_body(C, j, do_w3, do_next, do_idx):
        rb, isl = j % 3, j % 6
        gwait(rb, isl)
        sissue(rb, isl)
        if do_w3:
            swait((j + 1) % 3)
        if do_next:
            idx_wait(C + 2, (j + 2) % 6)
            gissue((j + 2) % 3, (j + 2) % 6)
        if do_idx:
            idx_load(C + 4, (j + 4) % 6)

    for t in range(4):
        idx_load(t, t)
    idx_wait(0, 0)
    idx_wait(1, 1)
    gissue(0, 0)
    gissue(1, 1)

    nblk = nchp // 6
    for j in range(6):
        chunk_body(j, j, do_w3=(j >= 2), do_next=True, do_idx=True)

    def body(i, carry):
        for j in range(6):
            chunk_body(i * 6 + j, j, True, True, True)
        return carry

    lax.fori_loop(1, nblk - 1, body, 0)

    C0 = (nblk - 1) * 6
    for j in range(6):
        chunk_body(C0 + j, j, True, C0 + j + 2 < nchp, C0 + j + 4 < nchp)
    swait((nchp - 2) % 3)
    swait((nchp - 1) % 3)


def _seg_scratch():
    return ([pltpu.VMEM((2, KE), jnp.int32) for _ in range(6)]
            + [pltpu.VMEM((KE, 128), jnp.float32) for _ in range(3)]
            + [pltpu.VMEM_SHARED((NP, 128), jnp.float32)]
            + [pltpu.SemaphoreType.DMA for _ in range(12)])


def _unpack_seg_scratch(scr):
    ibufs = scr[:6]
    rows = scr[6:9]
    acc = scr[9]
    isems = scr[10:16]
    gsems = scr[16:19]
    ssems = scr[19:22]
    return ibufs, rows, acc, isems, gsems, ssems


@functools.cache
def _make_segsum_edge_split(nchp):
    @functools.partial(
        pl.kernel,
        out_type=(jax.ShapeDtypeStruct((NP, 128), jnp.float32),
                  jax.ShapeDtypeStruct((NP, 128), jnp.float32)),
        mesh=_sc_mesh(),
        scratch_types=_seg_scratch(),
    )
    def seg(x, e3, zrows, out0, out1, *scr):
        ibufs, rows, acc, isems, gsems, ssems = _unpack_seg_scratch(scr)
        c = lax.axis_index("c")
        s = lax.axis_index("s")
        w = c * NS + s
        pltpu.sync_copy(zrows, acc.at[pl.ds(s * RPT, RPT)])
        plsc.subcore_barrier()
        _seg_pipeline([(c >= 0, x)], e3.at[w], acc, ibufs, rows,
                      isems, gsems, ssems, nchp)
        plsc.subcore_barrier()

        @pl.when(c == 0)
        def _():
            pltpu.sync_copy(acc.at[pl.ds(s * RPT, RPT)], out0.at[pl.ds(s * RPT, RPT)])

        @pl.when(c == 1)
        def _():
            pltpu.sync_copy(acc.at[pl.ds(s * RPT, RPT)], out1.at[pl.ds(s * RPT, RPT)])

    return seg


@functools.cache
def _make_segsum_feat_split(nchp):
    @functools.partial(
        pl.kernel,
        out_type=(jax.ShapeDtypeStruct((NP, 128), jnp.float32),
                  jax.ShapeDtypeStruct((NP, 128), jnp.float32)),
        mesh=_sc_mesh(),
        scratch_types=_seg_scratch(),
    )
    def seg(xa, xb, e3, zrows, outa, outb, *scr):
        ibufs, rows, acc, isems, gsems, ssems = _unpack_seg_scratch(scr)
        c = lax.axis_index("c")
        s = lax.axis_index("s")
        pltpu.sync_copy(zrows, acc.at[pl.ds(s * RPT, RPT)])
        plsc.subcore_barrier()
        _seg_pipeline([(c == 0, xa), (c == 1, xb)], e3.at[s], acc, ibufs,
                      rows, isems, gsems, ssems, nchp)
        plsc.subcore_barrier()

        @pl.when(c == 0)
        def _():
            pltpu.sync_copy(acc.at[pl.ds(s * RPT, RPT)], outa.at[pl.ds(s * RPT, RPT)])

        @pl.when(c == 1)
        def _():
            pltpu.sync_copy(acc.at[pl.ds(s * RPT, RPT)], outb.at[pl.ds(s * RPT, RPT)])

    return seg


@functools.cache
def _make_counts(nchp):
    @functools.partial(
        pl.kernel,
        out_type=(jax.ShapeDtypeStruct((NP, 128), jnp.float32),
                  jax.ShapeDtypeStruct((NP, 128), jnp.float32)),
        mesh=_sc_mesh(),
        scratch_types=(
            [pltpu.VMEM((nchp, KE), jnp.int32),
             pltpu.VMEM((KE, 128), jnp.float32),
             pltpu.VMEM_SHARED((NP, 128), jnp.float32)]
            + [pltpu.SemaphoreType.DMA for _ in range(4)]
        ),
    )
    def cnt(dw2, dr2, ones_hbm, zrows, outw, outr, idx_all, ones_v, acc, *ssems):
        c = lax.axis_index("c")
        s = lax.axis_index("s")
        pltpu.sync_copy(ones_hbm, ones_v)
        pltpu.sync_copy(zrows, acc.at[pl.ds(s * RPT, RPT)])

        @pl.when(c == 0)
        def _():
            pltpu.sync_copy(dw2.at[s], idx_all)

        @pl.when(c == 1)
        def _():
            pltpu.sync_copy(dr2.at[s], idx_all)

        plsc.subcore_barrier()

        def body(i, carry):
            for b in range(4):
                @pl.when(i > 0)
                def _():
                    pltpu.make_async_copy(ones_v, acc.at[idx_all.at[0]],
                                          ssems[b]).wait()
                pltpu.async_copy(ones_v, acc.at[idx_all.at[i * 4 + b]],
                                 ssems[b], add=True)
            return carry

        lax.fori_loop(0, nchp // 4, body, 0)
        for b in range(4):
            pltpu.make_async_copy(ones_v, acc.at[idx_all.at[0]], ssems[b]).wait()
        plsc.subcore_barrier()

        @pl.when(c == 0)
        def _():
            pltpu.sync_copy(acc.at[pl.ds(s * RPT, RPT)], outw.at[pl.ds(s * RPT, RPT)])

        @pl.when(c == 1)
        def _():
            pltpu.sync_copy(acc.at[pl.ds(s * RPT, RPT)], outr.at[pl.ds(s * RPT, RPT)])

    return cnt


@functools.cache
def _make_decoder_gather(BP, KD):
    per_tile = BP // (NC * NS)
    nchd = per_tile // KD
    assert nchd % 2 == 0

    @functools.partial(
        pl.kernel,
        out_type=tuple(jax.ShapeDtypeStruct((BP, 128), jnp.float32) for _ in range(4)),
        mesh=_sc_mesh(),
        scratch_types=(
            [pltpu.VMEM((nchd, 2, KD), jnp.int32)]
            + [pltpu.VMEM((KD, 128), jnp.float32) for _ in range(8)]
            + [pltpu.SemaphoreType.DMA for _ in range(16)]
        ),
    )
    def dec(hua, hub, hma, hmb, d3, gua, gub, gma, gmb, *scr):
        idx_all = scr[0]
        bufs = scr[1:9]
        rows = [bufs[:4], bufs[4:]]
        sems = scr[9:]
        gsems = [sems[:4], sems[4:8]]
        wsems = [sems[8:12], sems[12:]]
        tables = (hua, hub, hma, hmb)
        outs = (gua, gub, gma, gmb)
        c = lax.axis_index("c")
        s = lax.axis_index("s")
        w = c * NS + s
        pltpu.sync_copy(d3.at[w], idx_all)

        def gather(ch, grp):
            for t in range(4):
                pltpu.async_copy(tables[t].at[idx_all.at[ch, t // 2]],
                                 rows[grp][t], gsems[grp][t])

        gather(0, 0)

        def outer(i, carry):
            for p in (0, 1):
                r = 2 * i + p
                q = 1 - p
                base = w * per_tile + r * KD
                for t in range(4):
                    pltpu.make_async_copy(tables[t].at[idx_all.at[r, t // 2]],
                                          rows[p][t], gsems[p][t]).wait()
                    pltpu.async_copy(rows[p][t], outs[t].at[pl.ds(base, KD)],
                                     wsems[p][t])
                if p == 0:
                    @pl.when(i > 0)
                    def _():
                        for t in range(4):
                            pltpu.make_async_copy(rows[q][t], outs[t].at[pl.ds(0, KD)],
                                                  wsems[q][t]).wait()
                else:
                    for t in range(4):
                        pltpu.make_async_copy(rows[q][t], outs[t].at[pl.ds(0, KD)],
                                              wsems[q][t]).wait()

                @pl.when(r + 1 < nchd)
                def _():
                    gather(r + 1, q)
            return carry

        lax.fori_loop(0, nchd // 2, outer, 0)
        for t in range(4):
            pltpu.make_async_copy(rows[1][t], outs[t].at[pl.ds(0, KD)],
                                  wsems[1][t]).wait()

    return dec


def _dot():
    return functools.partial(
        lax.dot_general,
        dimension_numbers=(((1,), (0,)), ((), ())),
        preferred_element_type=jnp.float32,
        precision=lax.Precision.HIGHEST,
    )


RB = 1024


def _finish(res, bias, oa, ob):
    res += bias[...]
    res = jnp.where(res >= 0.0, res, 0.01 * res)
    rid = lax.broadcasted_iota(jnp.int32, res.shape, 0) + pl.program_id(0) * RB
    res = jnp.where(rid < N, res, 0.0)
    oa[...] = res[:, :128]
    ob[...] = res[:, 128:]


def _sage1_body(s0, s1, cnt, xd, wl, wr, bias, oa, ob):
    dot = _dot()
    inv = 1.0 / jnp.maximum(cnt[:, 0:1], 1.0)
    mean = (s0[...] + s1[...]) * inv
    res = dot(mean, wl[...]) + dot(xd[...], wr[...])
    _finish(res, bias, oa, ob)


def _sage2_body(sa, sb, cnt, xda, xdb, wla, wlb, wra, wrb, bias, oa, ob):
    dot = _dot()
    inv = 1.0 / jnp.maximum(cnt[:, 0:1], 1.0)
    res = dot(sa[...] * inv, wla[...])
    res += dot(sb[...] * inv, wlb[...])
    res += dot(xda[...], wra[...])
    res += dot(xdb[...], wrb[...])
    _finish(res, bias, oa, ob)


_row = lambda i: (i, 0)
_full = lambda i: (0, 0)


@functools.cache
def _make_sage1():
    return pl.pallas_call(
        _sage1_body,
        grid=(NP // RB,),
        in_specs=[
            pl.BlockSpec((RB, 128), _row),
            pl.BlockSpec((RB, 128), _row),
            pl.BlockSpec((RB, 128), _row),
            pl.BlockSpec((RB, 128), _row),
            pl.BlockSpec((128, 256), _full),
            pl.BlockSpec((128, 256), _full),
            pl.BlockSpec((1, 256), _full),
        ],
        out_specs=[pl.BlockSpec((RB, 128), _row)] * 2,
        out_shape=[jax.ShapeDtypeStruct((NP, 128), jnp.float32)] * 2,
    )


@functools.cache
def _make_sage2():
    return pl.pallas_call(
        _sage2_body,
        grid=(NP // RB,),
        in_specs=[
            pl.BlockSpec((RB, 128), _row),
            pl.BlockSpec((RB, 128), _row),
            pl.BlockSpec((RB, 128), _row),
            pl.BlockSpec((RB, 128), _row),
            pl.BlockSpec((RB, 128), _row),
            pl.BlockSpec((128, 256), _full),
            pl.BlockSpec((128, 256), _full),
            pl.BlockSpec((128, 256), _full),
            pl.BlockSpec((128, 256), _full),
            pl.BlockSpec((1, 256), _full),
        ],
        out_specs=[pl.BlockSpec((RB, 128), _row)] * 2,
        out_shape=[jax.ShapeDtypeStruct((NP, 128), jnp.float32)] * 2,
    )


def _dot_body(gua, gub, gma, gmb, o):
    o[...] = jnp.sum(gua[...] * gma[...] + gub[...] * gmb[...], axis=1)


@functools.cache
def _make_pair_dot(BP):
    return pl.pallas_call(
        _dot_body,
        grid=(BP // RB,),
        in_specs=[pl.BlockSpec((RB, 128), _row)] * 4,
        out_specs=pl.BlockSpec((RB,), lambda i: (i,)),
        out_shape=jax.ShapeDtypeStruct((BP,), jnp.float32),
    )


def _pack_edges(eidx, parts, nchp):
    E = eidx.shape[1]
    nch = E // parts // KE
    s3 = eidx[0].reshape(parts, nch, 1, KE)
    d3 = eidx[1].reshape(parts, nch, 1, KE)
    e3 = jnp.concatenate([s3, d3], axis=2)
    if nchp > nch:
        pad = jnp.concatenate(
            [jnp.full((parts, nchp - nch, 1, KE), ZROW, jnp.int32),
             jnp.full((parts, nchp - nch, 1, KE), N, jnp.int32)], axis=2)
        e3 = jnp.concatenate([e3, pad], axis=1)
    return e3


def _pack_dst(drow, parts, nchp):
    E = drow.shape[0]
    nch = E // parts // KE
    d2 = drow.reshape(parts, nch, KE)
    if nchp > nch:
        d2 = jnp.concatenate(
            [d2, jnp.full((parts, nchp - nch, KE), N, jnp.int32)], axis=1)
    return d2


def kernel(x_user, x_movie, edge_index_watched, edge_index_rev, edge_label_index,
           Wl_um, Wr_um, b_um, Wl_mu, Wr_mu, b_mu,
           Wl2, Wr2, b2, Wl3, Wr3, b3):
    f32 = jnp.float32
    E = edge_index_watched.shape[1]

    def _round_up(n, m):
        return -(-n // m) * m

    nch1 = _round_up(E // (NC * NS) // KE, 6)
    nch2 = _round_up(E // NS // KE, 6)
    nchc = _round_up(E // NS // KE, 4)

    ew = edge_index_watched.astype(jnp.int32)
    er = edge_index_rev.astype(jnp.int32)
    ew1 = _pack_edges(ew, NC * NS, nch1)
    er1 = _pack_edges(er, NC * NS, nch1)
    ew2 = _pack_edges(ew, NS, nch2)
    er2 = _pack_edges(er, NS, nch2)
    dw2 = _pack_dst(ew[1], NS, nchc)
    dr2 = _pack_dst(er[1], NS, nchc)

    zr128 = jnp.zeros((RPT, 128), f32)
    ones128 = jnp.ones((KE, 128), f32)
    xup = jnp.pad(x_user, ((0, 16), (0, 0)))
    xmp = jnp.pad(x_movie, ((0, 16), (0, 0)))

    counts_w, counts_r = _make_counts(nchc)(dw2, dr2, ones128, zr128)

    seg1 = _make_segsum_edge_split(nch1)
    seg2 = _make_segsum_feat_split(nch2)
    sage1 = _make_sage1()
    sage2 = _make_sage2()

    sm0, sm1 = seg1(xup, ew1, zr128)
    su0, su1 = seg1(xmp, er1, zr128)
    hm_a, hm_b = sage1(sm0, sm1, counts_w, x_movie, Wl_um, Wr_um, b_um.reshape(1, 256))
    hu_a, hu_b = sage1(su0, su1, counts_r, x_user, Wl_mu, Wr_mu, b_mu.reshape(1, 256))

    sm_a, sm_b = seg2(hu_a, hu_b, ew2, zr128)
    su_a, su_b = seg2(hm_a, hm_b, er2, zr128)
    hm2_a, hm2_b = sage2(sm_a, sm_b, counts_w, hm_a, hm_b,
                         Wl2[:128], Wl2[128:], Wr2[:128], Wr2[128:],
                         b2.reshape(1, 256))
    hu2_a, hu2_b = sage2(su_a, su_b, counts_r, hu_a, hu_b,
                         Wl2[:128], Wl2[128:], Wr2[:128], Wr2[128:],
                         b2.reshape(1, 256))

    sm_a, sm_b = seg2(hu2_a, hu2_b, ew2, zr128)
    su_a, su_b = seg2(hm2_a, hm2_b, er2, zr128)
    hm3_a, hm3_b = sage2(sm_a, sm_b, counts_w, hm2_a, hm2_b,
                         Wl3[:128], Wl3[128:], Wr3[:128], Wr3[128:],
                         b3.reshape(1, 256))
    hu3_a, hu3_b = sage2(su_a, su_b, counts_r, hu2_a, hu2_b,
                         Wl3[:128], Wl3[128:], Wr3[:128], Wr3[128:],
                         b3.reshape(1, 256))

    B = edge_label_index.shape[1]
    KD = 80
    BP = _round_up(B, NC * NS * KD * 2)
    eli = edge_label_index.astype(jnp.int32)
    u3 = jnp.pad(eli[0], (0, BP - B)).reshape(NC * NS, -1, 1, KD)
    m3 = jnp.pad(eli[1], (0, BP - B)).reshape(NC * NS, -1, 1, KD)
    d3 = jnp.concatenate([u3, m3], axis=2)
    gua, gub, gma, gmb = _make_decoder_gather(BP, KD)(hu3_a, hu3_b, hm3_a, hm3_b, d3)
    dots = _make_pair_dot(BP)(gua, gub, gma, gmb)
    return dots[:B]
